# baseline (speedup 1.0000x reference)
import functools

import numpy as np

import jax
import jax.numpy as jnp
from jax.experimental import pallas as pl
from jax.experimental.pallas import tpu as pltpu
from jax.sharding import Mesh, NamedSharding, PartitionSpec as P

try:
    from jax.experimental.shard_map import shard_map as _shard_map
except ImportError:
    from jax import shard_map as _shard_map


def _proj_kernel(x_ref, w_ref, b_ref, o_ref):
    x_bf = x_ref[...].astype(jnp.bfloat16)
    acc = jax.lax.dot_general(
        x_bf, w_ref[...],
        dimension_numbers=(((1,), (0,)), ((), ())),
        preferred_element_type=jnp.float32,
    )
    o_ref[...] = acc + b_ref[...]


@functools.partial(jax.jit, static_argnames=("bm",))
def _proj(x, wt, b2, bm):
    tot, enc = x.shape
    dec = wt.shape[1]
    return pl.pallas_call(
        _proj_kernel,
        grid=(tot // bm,),
        in_specs=[
            pl.BlockSpec((bm, enc), lambda i: (i, 0)),
            pl.BlockSpec((enc, dec), lambda i: (0, 0)),
            pl.BlockSpec((1, dec), lambda i: (0, 0)),
        ],
        out_specs=pl.BlockSpec((bm, dec), lambda i: (i, 0)),
        out_shape=jax.ShapeDtypeStruct((tot, dec), jnp.float32),
        compiler_params=pltpu.CompilerParams(
            dimension_semantics=("arbitrary",),
            vmem_limit_bytes=100 * 1024 * 1024,
        ),
    )(x, wt, b2)


def kernel(x, W, b):
    wt = W.astype(jnp.bfloat16).T
    b2 = b[None, :]
    devs = jax.devices()
    nd = len(devs)
    while nd > 1 and x.shape[0] % (nd * 1024) != 0:
        nd -= 1
    if nd <= 1:
        return _proj(x, wt, b2, bm=1024)
    mesh = Mesh(np.asarray(devs[:nd]), ("d",))
    xs = jax.device_put(x, NamedSharding(mesh, P("d", None)))
    wr = jax.device_put(wt, NamedSharding(mesh, P(None, None)))
    br = jax.device_put(b2, NamedSharding(mesh, P(None, None)))
    f = _shard_map(
        lambda xc, w, bb: _proj(xc, w, bb, bm=1024),
        mesh=mesh,
        in_specs=(P("d", None), P(None, None), P(None, None)),
        out_specs=P("d", None),
        check_rep=False,
    )
    return f(xs, wr, br)


# speedup vs baseline: 2.2111x; 2.2111x over previous
import functools

import jax
import jax.numpy as jnp
from jax.experimental import pallas as pl
from jax.experimental.pallas import tpu as pltpu


def _proj_kernel(x_ref, w_ref, b_ref, o_ref):
    x_bf = x_ref[...].astype(jnp.bfloat16)
    acc = jax.lax.dot_general(
        x_bf, w_ref[...],
        dimension_numbers=(((1,), (0,)), ((), ())),
        preferred_element_type=jnp.float32,
    )
    o_ref[...] = acc + b_ref[...]


@functools.partial(jax.jit, static_argnames=("bm",))
def _proj(x, wt, b2, bm):
    tot, enc = x.shape
    dec = wt.shape[1]
    return pl.pallas_call(
        _proj_kernel,
        grid=(tot // bm,),
        in_specs=[
            pl.BlockSpec((bm, enc), lambda i: (i, 0)),
            pl.BlockSpec((enc, dec), lambda i: (0, 0)),
            pl.BlockSpec((1, dec), lambda i: (0, 0)),
        ],
        out_specs=pl.BlockSpec((bm, dec), lambda i: (i, 0)),
        out_shape=jax.ShapeDtypeStruct((tot, dec), jnp.float32),
        compiler_params=pltpu.CompilerParams(
            dimension_semantics=("parallel",),
            vmem_limit_bytes=100 * 1024 * 1024,
        ),
    )(x, wt, b2)


def kernel(x, W, b):
    wt = W.astype(jnp.bfloat16).T
    b2 = b[None, :]
    return _proj(x, wt, b2, bm=1024)


# speedup vs baseline: 2.2188x; 1.0035x over previous
import functools

import jax
import jax.numpy as jnp
from jax.experimental import pallas as pl
from jax.experimental.pallas import tpu as pltpu


def _proj_kernel(x_ref, w_ref, b_ref, o_ref):
    x_bf = x_ref[...].astype(jnp.bfloat16)
    acc = jax.lax.dot_general(
        x_bf, w_ref[...],
        dimension_numbers=(((1,), (1,)), ((), ())),
        preferred_element_type=jnp.float32,
    )
    o_ref[...] = acc + b_ref[...]


@functools.partial(jax.jit, static_argnames=("bm",))
def _proj(x, wb, b2, bm):
    tot, enc = x.shape
    dec = wb.shape[0]
    return pl.pallas_call(
        _proj_kernel,
        grid=(tot // bm,),
        in_specs=[
            pl.BlockSpec((bm, enc), lambda i: (i, 0)),
            pl.BlockSpec((dec, enc), lambda i: (0, 0)),
            pl.BlockSpec((1, dec), lambda i: (0, 0)),
        ],
        out_specs=pl.BlockSpec((bm, dec), lambda i: (i, 0)),
        out_shape=jax.ShapeDtypeStruct((tot, dec), jnp.float32),
        compiler_params=pltpu.CompilerParams(
            dimension_semantics=("arbitrary",),
        ),
    )(x, wb, b2)


def kernel(x, W, b):
    wb = W.astype(jnp.bfloat16)
    b2 = b[None, :]
    return _proj(x, wb, b2, bm=1024)


# speedup vs baseline: 2.2700x; 1.0231x over previous
import functools

import jax
import jax.numpy as jnp
from jax.experimental import pallas as pl
from jax.experimental.pallas import tpu as pltpu


def _proj_kernel(x_ref, w_ref, b_ref, o_ref, wbf_ref):
    @pl.when(pl.program_id(0) == 0)
    def _():
        wbf_ref[...] = w_ref[...].astype(jnp.bfloat16)

    x_bf = x_ref[...].astype(jnp.bfloat16)
    acc = jax.lax.dot_general(
        x_bf, wbf_ref[...],
        dimension_numbers=(((1,), (1,)), ((), ())),
        preferred_element_type=jnp.float32,
    )
    o_ref[...] = acc + b_ref[...]


@functools.partial(jax.jit, static_argnames=("bm",))
def _proj(x, w, b2, bm):
    tot, enc = x.shape
    dec = w.shape[0]
    return pl.pallas_call(
        _proj_kernel,
        grid=(tot // bm,),
        in_specs=[
            pl.BlockSpec((bm, enc), lambda i: (i, 0)),
            pl.BlockSpec((dec, enc), lambda i: (0, 0)),
            pl.BlockSpec((1, dec), lambda i: (0, 0)),
        ],
        out_specs=pl.BlockSpec((bm, dec), lambda i: (i, 0)),
        out_shape=jax.ShapeDtypeStruct((tot, dec), jnp.float32),
        scratch_shapes=[pltpu.VMEM((dec, enc), jnp.bfloat16)],
        compiler_params=pltpu.CompilerParams(
            dimension_semantics=("arbitrary",),
            vmem_limit_bytes=100 * 1024 * 1024,
        ),
    )(x, w, b2)


def kernel(x, W, b):
    b2 = b[None, :]
    return _proj(x, W, b2, bm=1024)
